# baseline (speedup 1.0000x reference)
import functools

import jax
import jax.numpy as jnp
from jax import lax
from jax.experimental import pallas as pl
from jax.experimental.pallas import tpu as pltpu
from jax.experimental.pallas import tpu_sc as plsc


def _tables_body(x_ref, wab_ref, out_ref):
    x = x_ref[...]
    dn = (((1,), (1,)), ((), ()))
    out_ref[0] = lax.dot_general(x, wab_ref[0], dn,
                                 preferred_element_type=jnp.float32)
    out_ref[1] = lax.dot_general(x, wab_ref[1], dn,
                                 preferred_element_type=jnp.float32)


def _node_tables(x, w1, w2):
    n, d = x.shape
    wab = jnp.stack([w1, w2])
    return pl.pallas_call(
        _tables_body,
        out_shape=jax.ShapeDtypeStruct((2, n, d), jnp.float32),
    )(x, wab)


def _make_sc_gather_pack(e, d, chunk, nc, ns):
    epw = e // (nc * ns)
    nchunks = epw // chunk
    dw = d // 2
    mesh = plsc.VectorSubcoreMesh(core_axis_name="c", subcore_axis_name="s")
    gbuf = lambda: pltpu.VMEM((chunk, d), jnp.float32)

    @functools.partial(
        pl.kernel,
        out_type=jax.ShapeDtypeStruct((e * dw,), jnp.int32),
        mesh=mesh,
        scratch_types=[
            pltpu.VMEM((epw,), jnp.int32),
            pltpu.VMEM((epw,), jnp.int32),
            gbuf(), gbuf(),
            pltpu.VMEM((chunk * dw,), jnp.int32),
            gbuf(), gbuf(),
            pltpu.VMEM((chunk * dw,), jnp.int32),
            pltpu.SemaphoreType.DMA,
            pltpu.SemaphoreType.DMA,
            pltpu.SemaphoreType.DMA,
            pltpu.SemaphoreType.DMA,
        ],
    )
    def sc_kernel(table_hbm, src_hbm, dst_hbm, g_hbm,
                  idx_s, idx_d,
                  s0, d0, o0, s1, d1, o1,
                  sem0, sem1, semw0, semw1):
        wid = lax.axis_index("s") * nc + lax.axis_index("c")
        base_w = wid * epw
        pltpu.sync_copy(src_hbm.at[pl.ds(base_w, epw)], idx_s)
        pltpu.sync_copy(dst_hbm.at[pl.ds(base_w, epw)], idx_d)

        sets = ((s0, d0, o0, sem0, semw0), (s1, d1, o1, sem1, semw1))

        def in_copies(c, st):
            bs, bd, _, sem, _ = st
            off = c * chunk
            return (
                pltpu.make_async_copy(
                    table_hbm.at[idx_s.at[pl.ds(off, chunk)]], bs, sem),
                pltpu.make_async_copy(
                    table_hbm.at[idx_d.at[pl.ds(off, chunk)]], bd, sem),
            )

        def rnd(x):
            b = lax.bitcast_convert_type(x, jnp.int32)
            return b + jnp.int32(0x8000)

        def compute(st):
            bs, bd, bo, _, _ = st

            def row_body(r, carry):
                for i in range(d // 32):
                    sl = pl.ds(16 * i, 16)
                    sh = pl.ds(dw + 16 * i, 16)
                    lo = rnd(bs[r, sl] + bd[r, sl])
                    hi = rnd(bs[r, sh] + bd[r, sh])
                    word = (
                        (hi & jnp.int32(-65536))
                        | ((lo >> 16) & jnp.int32(0xFFFF))
                    )
                    bo[pl.ds(r * dw + 16 * i, 16)] = word
                return carry

            lax.fori_loop(0, chunk, row_body, 0, unroll=False)

        def store(c, st):
            _, _, bo, _, semw = st
            return pltpu.make_async_copy(
                bo, g_hbm.at[pl.ds((base_w + c * chunk) * dw, chunk * dw)],
                semw)

        for cp in in_copies(0, sets[0]):
            cp.start()

        def pair_body(i, carry):
            ca = 2 * i
            cb = 2 * i + 1
            for cp in in_copies(cb, sets[1]):
                cp.start()
            for cp in in_copies(ca, sets[0]):
                cp.wait()

            @pl.when(i > 0)
            def _():
                store(ca, sets[0]).wait()

            compute(sets[0])
            store(ca, sets[0]).start()

            @pl.when(cb + 1 < nchunks)
            def _():
                for cp in in_copies(cb + 1, sets[0]):
                    cp.start()

            for cp in in_copies(cb, sets[1]):
                cp.wait()

            @pl.when(i > 0)
            def _():
                store(cb, sets[1]).wait()

            compute(sets[1])
            store(cb, sets[1]).start()
            return carry

        lax.fori_loop(0, nchunks // 2, pair_body, 0, unroll=False)
        store(nchunks - 2, sets[0]).wait()
        store(nchunks - 1, sets[1]).wait()

    return sc_kernel


def _final_body(ea_ref, w3_ref, g_ref, out_ref):
    dn = (((1,), (1,)), ((), ()))
    mm = lax.dot_general(ea_ref[...], w3_ref[...], dn,
                         preferred_element_type=jnp.float32)
    w = g_ref[...]
    lo = lax.bitcast_convert_type(w << 16, jnp.float32)
    hi = lax.bitcast_convert_type(w & jnp.int32(-65536), jnp.float32)
    out_ref[...] = mm + jnp.concatenate([lo, hi], axis=1)


def _final(edge_attr, w3, g, block_e):
    e, d = edge_attr.shape
    grid = (e // block_e,)
    return pl.pallas_call(
        _final_body,
        grid=grid,
        in_specs=[
            pl.BlockSpec((block_e, d), lambda i: (i, 0)),
            pl.BlockSpec((d, d), lambda i: (0, 0)),
            pl.BlockSpec((block_e, d // 2), lambda i: (i, 0)),
        ],
        out_specs=pl.BlockSpec((block_e, d), lambda i: (i, 0)),
        out_shape=jax.ShapeDtypeStruct((e, d), jnp.float32),
    )(edge_attr, w3, g)


def kernel(x, edge_index, edge_attr, W):
    n, d = x.shape
    e = edge_attr.shape[0]
    w1, w2, w3 = W[:, :d], W[:, d:2 * d], W[:, 2 * d:]

    tables = _node_tables(x, w1, w2).reshape(2 * n, d)

    src = edge_index[0]
    dstn = edge_index[1] + n

    nc, ns = 2, 16
    chunk = 40
    sc = _make_sc_gather_pack(e, d, chunk, nc, ns)
    g = sc(tables, src, dstn).reshape(e, d // 2)

    return _final(edge_attr, w3, g, block_e=4000)


# speedup vs baseline: 1.9356x; 1.9356x over previous
import functools

import jax
import jax.numpy as jnp
from jax import lax
from jax.experimental import pallas as pl
from jax.experimental.pallas import tpu as pltpu
from jax.experimental.pallas import tpu_sc as plsc


def _tables_body(x_ref, wab_ref, out_ref):
    x = x_ref[...]
    dn = (((1,), (1,)), ((), ()))
    out_ref[0] = lax.dot_general(x, wab_ref[0], dn,
                                 preferred_element_type=jnp.float32)
    out_ref[1] = lax.dot_general(x, wab_ref[1], dn,
                                 preferred_element_type=jnp.float32)


def _node_tables(x, w1, w2):
    n, d = x.shape
    wab = jnp.stack([w1, w2])
    return pl.pallas_call(
        _tables_body,
        out_shape=jax.ShapeDtypeStruct((2, n, d), jnp.float32),
    )(x, wab)


def _make_sc_gather_pack(e, d, chunk, nc, ns):
    nw = nc * ns
    eh = e // 2
    rpw = eh // nw
    nchunks = rpw // chunk
    mesh = plsc.VectorSubcoreMesh(core_axis_name="c", subcore_axis_name="s")
    gbuf = lambda: pltpu.VMEM((chunk, d), jnp.float32)
    ibuf = lambda: pltpu.VMEM((rpw,), jnp.int32)

    @functools.partial(
        pl.kernel,
        out_type=jax.ShapeDtypeStruct((eh, d), jnp.int32),
        mesh=mesh,
        scratch_types=[
            ibuf(), ibuf(), ibuf(), ibuf(),
            gbuf(), gbuf(), gbuf(), gbuf(),
            pltpu.VMEM((chunk, d), jnp.int32),
            gbuf(), gbuf(), gbuf(), gbuf(),
            pltpu.VMEM((chunk, d), jnp.int32),
            pltpu.SemaphoreType.DMA,
            pltpu.SemaphoreType.DMA,
            pltpu.SemaphoreType.DMA,
            pltpu.SemaphoreType.DMA,
        ],
    )
    def sc_kernel(table_hbm, src_hbm, dst_hbm, g_hbm,
                  isl, idl, ish, idh,
                  sl0, dl0, sh0, dh0, o0,
                  sl1, dl1, sh1, dh1, o1,
                  sem0, sem1, semw0, semw1):
        wid = lax.axis_index("s") * nc + lax.axis_index("c")
        base_lo = wid * rpw
        base_hi = eh + base_lo
        pltpu.sync_copy(src_hbm.at[pl.ds(base_lo, rpw)], isl)
        pltpu.sync_copy(dst_hbm.at[pl.ds(base_lo, rpw)], idl)
        pltpu.sync_copy(src_hbm.at[pl.ds(base_hi, rpw)], ish)
        pltpu.sync_copy(dst_hbm.at[pl.ds(base_hi, rpw)], idh)

        sets = ((sl0, dl0, sh0, dh0, o0, sem0, semw0),
                (sl1, dl1, sh1, dh1, o1, sem1, semw1))

        def in_copies(c, st):
            bsl, bdl, bsh, bdh, _, sem, _ = st
            off = c * chunk
            mk = pltpu.make_async_copy
            return (
                mk(table_hbm.at[isl.at[pl.ds(off, chunk)]], bsl, sem),
                mk(table_hbm.at[idl.at[pl.ds(off, chunk)]], bdl, sem),
                mk(table_hbm.at[ish.at[pl.ds(off, chunk)]], bsh, sem),
                mk(table_hbm.at[idh.at[pl.ds(off, chunk)]], bdh, sem),
            )

        def rnd(x):
            b = lax.bitcast_convert_type(x, jnp.int32)
            return b + jnp.int32(0x8000)

        def compute(st):
            bsl, bdl, bsh, bdh, bo, _, _ = st

            def row_body(r, carry):
                for i in range(d // 16):
                    sl = pl.ds(16 * i, 16)
                    lo = rnd(bsl[r, sl] + bdl[r, sl])
                    hi = rnd(bsh[r, sl] + bdh[r, sl])
                    bo[r, sl] = (
                        (hi & jnp.int32(-65536))
                        | ((lo >> 16) & jnp.int32(0xFFFF))
                    )
                return carry

            lax.fori_loop(0, chunk, row_body, 0, unroll=False)

        def start_in(c, st):
            for cp in in_copies(c, st):
                cp.start()

        def wait_in(c, st):
            for cp in in_copies(c, st):
                cp.wait()

        def store(c, st):
            bo, semw = st[4], st[6]
            return pltpu.make_async_copy(
                bo, g_hbm.at[pl.ds(wid * rpw + c * chunk, chunk)], semw)

        start_in(0, sets[0])
        start_in(1, sets[1])
        wait_in(0, sets[0])
        compute(sets[0])
        store(0, sets[0]).start()
        start_in(2, sets[0])

        def pair_body(i, carry):
            cb = 2 * i + 1
            ca = 2 * i + 2
            wait_in(cb, sets[1])

            @pl.when(i > 0)
            def _():
                store(cb, sets[1]).wait()

            compute(sets[1])
            store(cb, sets[1]).start()

            @pl.when(cb + 2 < nchunks)
            def _():
                start_in(cb + 2, sets[1])

            wait_in(ca, sets[0])
            store(ca, sets[0]).wait()
            compute(sets[0])
            store(ca, sets[0]).start()

            @pl.when(ca + 2 < nchunks)
            def _():
                start_in(ca + 2, sets[0])

            return carry

        lax.fori_loop(0, (nchunks - 1) // 2, pair_body, 0, unroll=False)
        store(nchunks - 2, sets[1]).wait()
        store(nchunks - 1, sets[0]).wait()

    return sc_kernel


def _final_body(ea_ref, w3_ref, g_ref, out_ref):
    dn = (((1,), (1,)), ((), ()))
    w = g_ref[...]
    lo = lax.bitcast_convert_type(w << 16, jnp.float32)
    hi = lax.bitcast_convert_type(w & jnp.int32(-65536), jnp.float32)
    out_ref[0] = lo + lax.dot_general(ea_ref[0], w3_ref[...], dn,
                                      preferred_element_type=jnp.float32)
    out_ref[1] = hi + lax.dot_general(ea_ref[1], w3_ref[...], dn,
                                      preferred_element_type=jnp.float32)


def _final(ea2, w3, g, block_e):
    _, eh, d = ea2.shape
    grid = (eh // block_e,)
    return pl.pallas_call(
        _final_body,
        grid=grid,
        in_specs=[
            pl.BlockSpec((2, block_e, d), lambda i: (0, i, 0)),
            pl.BlockSpec((d, d), lambda i: (0, 0)),
            pl.BlockSpec((block_e, d), lambda i: (i, 0)),
        ],
        out_specs=pl.BlockSpec((2, block_e, d), lambda i: (0, i, 0)),
        out_shape=jax.ShapeDtypeStruct((2, eh, d), jnp.float32),
    )(ea2, w3, g)


def kernel(x, edge_index, edge_attr, W):
    n, d = x.shape
    e = edge_attr.shape[0]
    w1, w2, w3 = W[:, :d], W[:, d:2 * d], W[:, 2 * d:]

    tables = _node_tables(x, w1, w2).reshape(2 * n, d)

    src = edge_index[0]
    dstn = edge_index[1] + n

    nc, ns = 2, 16
    chunk = 40
    sc = _make_sc_gather_pack(e, d, chunk, nc, ns)
    g = sc(tables, src, dstn)

    ea2 = edge_attr.reshape(2, e // 2, d)
    out2 = _final(ea2, w3, g, block_e=4000)
    return out2.reshape(e, d)
